# baseline (speedup 1.0000x reference)
import functools

import jax
import jax.numpy as jnp
from jax import lax
from jax.experimental import pallas as pl
from jax.experimental.pallas import tpu as pltpu
from jax.experimental.pallas import tpu_sc as plsc

L = 16
NROWS = 128
NCOLS = 32768
NW = 32
RPW = NROWS // NW
BIG = 2**30


_GATHER_DNUMS = lax.GatherDimensionNumbers(
    offset_dims=(), collapsed_slice_dims=(0,), start_index_map=(0,))


def _permute(x, perm):
    return lax.gather(x, perm[:, None], _GATHER_DNUMS, (1,),
                      mode=lax.GatherScatterMode.PROMISE_IN_BOUNDS)


def _all_reduce(x, lane, op):
    for s in (1, 2, 4, 8):
        x = op(x, _permute(x, lane ^ s))
    return x


def _finalize_row(m1, i1, m2, i2, lane):
    v1 = _all_reduce(m1, lane, jnp.maximum)
    i1g = _all_reduce(jnp.where(m1 == v1, i1, BIG), lane, jnp.minimum)
    winner = (m1 == v1) & (i1 == i1g)
    c = jnp.where(winner, m2, m1)
    ci = jnp.where(winner, i2, i1)
    v2 = _all_reduce(c, lane, jnp.maximum)
    i2g = _all_reduce(jnp.where(c == v2, ci, BIG), lane, jnp.minimum)
    return v1, i1g, v2, i2g


def _make_sc_kernel():
    mesh = plsc.VectorSubcoreMesh(core_axis_name="c", subcore_axis_name="s")

    @functools.partial(
        pl.kernel,
        out_type=(
            jax.ShapeDtypeStruct((NROWS * 2,), jnp.float32),
            jax.ShapeDtypeStruct((NROWS * 2,), jnp.int32),
        ),
        mesh=mesh,
        scratch_types=[
            pltpu.VMEM((NCOLS,), jnp.float32),
            pltpu.VMEM((NCOLS,), jnp.float32),
            pltpu.VMEM((L,), jnp.float32),
            pltpu.VMEM((L,), jnp.int32),
            pltpu.SemaphoreType.DMA,
            pltpu.SemaphoreType.DMA,
        ],
    )
    def topk2(var_hbm, outv_hbm, outi_hbm, buf0, buf1, resv_ref, resi_ref,
              sem0, sem1):
        wid = lax.axis_index("c") * 16 + lax.axis_index("s")
        base_row = wid * RPW
        lane = lax.broadcasted_iota(jnp.int32, (L,), 0)
        neg = jnp.full((L,), -jnp.inf, jnp.float32)
        zero_i = jnp.zeros((L,), jnp.int32)

        bufs = [buf0, buf1]
        sems = [sem0, sem1]
        cps = [None, None]
        cps[0] = pltpu.async_copy(var_hbm.at[base_row], buf0, sem0)

        resv = jnp.zeros((L,), jnp.float32)
        resi = jnp.zeros((L,), jnp.int32)
        for r in range(RPW):
            b = r % 2
            if r + 1 < RPW:
                nb = (r + 1) % 2
                cps[nb] = pltpu.async_copy(
                    var_hbm.at[base_row + r + 1], bufs[nb], sems[nb])
            cps[b].wait()
            buf = bufs[b]

            def body(t, carry):
                m1, i1, m2, i2 = carry
                x = buf[pl.ds(t * L, L)]
                ix = lane + t * L
                gt1 = x > m1
                cand = jnp.where(gt1, m1, x)
                candi = jnp.where(gt1, i1, ix)
                m1 = jnp.where(gt1, x, m1)
                i1 = jnp.where(gt1, ix, i1)
                gt2 = cand > m2
                m2 = jnp.where(gt2, cand, m2)
                i2 = jnp.where(gt2, candi, i2)
                return m1, i1, m2, i2

            m1, i1, m2, i2 = lax.fori_loop(
                0, NCOLS // L, body, (neg, zero_i, neg, zero_i))
            v1, i1g, v2, i2g = _finalize_row(m1, i1, m2, i2, lane)
            resv = jnp.where(lane == 2 * r, v1, resv)
            resv = jnp.where(lane == 2 * r + 1, v2, resv)
            resi = jnp.where(lane == 2 * r, i1g, resi)
            resi = jnp.where(lane == 2 * r + 1, i2g, resi)

        resv_ref[...] = resv
        resi_ref[...] = resi
        pltpu.sync_copy(resv_ref.at[pl.ds(0, 2 * RPW)],
                        outv_hbm.at[pl.ds(base_row * 2, 2 * RPW)])
        pltpu.sync_copy(resi_ref.at[pl.ds(0, 2 * RPW)],
                        outi_hbm.at[pl.ds(base_row * 2, 2 * RPW)])

    return topk2


_topk2_sc = _make_sc_kernel()


@jax.jit
def kernel(var):
    v, i = _topk2_sc(var)
    return v.reshape(NROWS, 2), i.reshape(NROWS, 2)
